# baseline (speedup 1.0000x reference)
import functools

import jax
import jax.numpy as jnp
from jax import lax
from jax.experimental import pallas as pl
from jax.experimental.pallas import tpu as pltpu
from jax.experimental.pallas import tpu_sc as plsc

_V = 1000000
_D = 16
_B = 81920
_S = 16384
_NSLAB = 62
_BPT = _B // 16
_G = _BPT // 16

_info = plsc.get_sparse_core_info()
_NC = _info.num_cores
_NS = _info.num_subcores

_mesh = plsc.VectorSubcoreMesh(core_axis_name="c", subcore_axis_name="s")

_i32 = jnp.int32


def _lane_gather(x, i):
    dn = lax.GatherDimensionNumbers(
        offset_dims=(), collapsed_slice_dims=(0,), start_index_map=(0,)
    )
    return lax.gather(
        x, i[:, None], dimension_numbers=dn, slice_sizes=(1,),
        mode=lax.GatherScatterMode.PROMISE_IN_BOUNDS,
    )


@functools.partial(
    pl.kernel,
    mesh=_mesh,
    compiler_params=pltpu.CompilerParams(needs_layout_passes=False),
    out_type=jax.ShapeDtypeStruct((_D, 4 * _B), jnp.float32),
    scratch_types=[
        pltpu.VMEM((_BPT,), _i32),
        pltpu.VMEM((_BPT,), _i32),
        pltpu.VMEM((64,), _i32),
        pltpu.VMEM((64,), _i32),
        pltpu.VMEM((64,), _i32),
        pltpu.VMEM((512,), jnp.float32),
        pltpu.VMEM((_D, _BPT), jnp.float32),
        pltpu.VMEM((_D, 64), jnp.float32),
        pltpu.VMEM_SHARED((_D * _S,), jnp.float32),
        pltpu.VMEM_SHARED((_D * _S,), jnp.float32),
        pltpu.SemaphoreType.DMA,
        pltpu.SemaphoreType.DMA,
    ],
)
def _grouped_gather(i0, i1, i2, i3, w0, w1, w2, w3, t0, t1, t2, t3, out,
                    idx_v, binned, cnt, fill, starts, cbuf,
                    stage, tailb, slab_a, slab_b, gsem, fsem):
    cid = lax.axis_index("c")
    sid = lax.axis_index("s")
    iot = lax.iota(_i32, 16)

    def fire_fill(wh, r0, sz, buf, cond):
        for c in range(_D):
            @pl.when((sid == c) & cond)
            def _():
                pltpu.async_copy(
                    wh.at[c, pl.ds(r0, sz)], buf.at[pl.ds(c * _S, sz)], fsem)

    def wait_fill(wh, sz, buf, cond):
        @pl.when(cond)
        def _():
            pltpu.make_async_copy(
                wh.at[0, pl.ds(0, sz)], buf.at[pl.ds(0, sz)], fsem).wait()

    def seg_rank(keys):
        bins = lax.shift_right_logical(keys, 14)
        prev = _lane_gather(bins, jnp.maximum(iot - 1, 0))
        newrun = (bins != prev) | (iot == 0)
        runstart = plsc.cummax(jnp.where(newrun, iot, 0))
        rank = iot - runstart
        flagv = jnp.where(newrun, 1, 0)
        nxt = _lane_gather(flagv, jnp.minimum(iot + 1, 15))
        last = (nxt == 1) | (iot == 15)
        return bins, rank, last

    def _bin_bounds(s):
        lane = s % 16
        half = s // 16
        sv = starts[pl.ds(0, 16)]
        cv = cnt[pl.ds(0, 16)]
        for h in range(1, 4):
            sv = jnp.where(half == h, starts[pl.ds(16 * h, 16)], sv)
            cv = jnp.where(half == h, cnt[pl.ds(16 * h, 16)], cv)
        st = jnp.sum(jnp.where(iot == lane, sv, 0))
        cn = jnp.sum(jnp.where(iot == lane, cv, 0))
        st = jnp.clip(st, 0, _BPT - 1)
        cn = jnp.clip(cn, 0, _BPT - st)
        return st, cn

    def gather_slab(s, buf, wait_handles):
        st, cn = _bin_bounds(s)
        r0 = s * _S
        ng = (cn + 15) // 16

        def fire(u):
            off = st + u * 16
            pk = binned[pl.ds(off, 16)]
            lr0 = pk & (_S - 1)
            msk = (iot < (cn - u * 16)) & ((r0 + lr0) < _V - 64)
            lr = jnp.where(msk, lr0, 0)
            half = (u & 1) * 256
            for c in range(_D):
                pltpu.async_copy(
                    buf.at[lr + c * _S],
                    cbuf.at[pl.ds(half + c * 16, 16)], gsem)

        @pl.when(ng > 0)
        def _():
            fire(jnp.int32(0))

        def grp(u, _):
            @pl.when(u + 1 < ng)
            def _():
                fire(u + 1)
            pltpu.make_async_copy(
                buf.at[pl.ds(0, 256)], cbuf.at[pl.ds(0, 256)], gsem).wait()
            off = st + u * 16
            pk = binned[pl.ds(off, 16)]
            bpvec = lax.shift_right_logical(pk, 14)
            lr0 = pk & (_S - 1)
            msk = (iot < (cn - u * 16)) & ((r0 + lr0) < _V - 64)
            half = (u & 1) * 256
            for c in range(_D):
                vals = cbuf[pl.ds(half + c * 16, 16)]
                plsc.store_scatter(
                    stage, [jnp.full((16,), c, _i32), bpvec], vals, mask=msk)
            return ()

        lax.fori_loop(0, ng, grp, ())
        for h in wait_handles:
            h.wait()

    def process(ih, wh, wt, slot):
        plsc.subcore_barrier()
        pltpu.sync_copy(ih.at[pl.ds(sid * _BPT, _BPT)], idx_v)
        pltpu.sync_copy(wt, tailb)
        z16 = jnp.zeros((16,), _i32)
        for h in range(4):
            cnt[pl.ds(16 * h, 16)] = z16
            fill[pl.ds(16 * h, 16)] = z16

        rprobe, _unused = plsc.scan_count(jnp.zeros((16,), _i32))
        rbase = jnp.sum(jnp.where(iot == 0, rprobe, 0))

        def pass_a(g, _):
            off = g * 16
            rv = idx_v[pl.ds(off, 16)]
            bins = lax.shift_right_logical(rv, 14)
            rank, last = plsc.scan_count(bins)
            rank = rank - rbase
            plsc.addupdate_scatter(cnt, [bins], rank + 1, mask=last)
            return ()

        lax.fori_loop(0, _G, pass_a, ())

        carry = jnp.int32(0)
        for h in range(4):
            ch = cnt[pl.ds(16 * h, 16)]
            inh = plsc.cumsum(ch) + carry
            starts[pl.ds(16 * h, 16)] = inh - ch
            carry = carry + jnp.sum(ch)

        def pass_b(g, _):
            off = g * 16
            rv = idx_v[pl.ds(off, 16)]
            bp = iot + off
            bins = lax.shift_right_logical(rv, 14)
            rank, last = plsc.scan_count(bins)
            rank = rank - rbase
            stv = plsc.load_gather(starts, [bins])
            fl = plsc.load_gather(fill, [bins])
            pos = jnp.clip(stv + fl + rank, 0, _BPT - 1)
            pk = lax.shift_left(bp, 14) | (rv & (_S - 1))
            plsc.store_scatter(binned, [pos], pk)
            plsc.addupdate_scatter(fill, [bins], rank + 1, mask=last)
            return ()

        lax.fori_loop(0, _G, pass_b, ())

        tcond = jnp.bool_(True)
        fire_fill(wh, 0, _S, slab_a, tcond)
        wait_fill(wh, _S, slab_a, tcond)
        plsc.subcore_barrier()

        def pair(u, _):
            r0b = pl.multiple_of((2 * u + 1) * _S, 128)
            fire_fill(wh, r0b, _S, slab_b, tcond)
            gather_slab(2 * u, slab_a, ())
            wait_fill(wh, _S, slab_b, tcond)
            plsc.subcore_barrier()
            s_a = 2 * u + 2
            do_fill = s_a <= 60
            r0a = pl.multiple_of(s_a * _S, 128)
            fire_fill(wh, r0a, _S, slab_a, do_fill)
            gather_slab(2 * u + 1, slab_b, ())
            wait_fill(wh, _S, slab_a, do_fill)
            plsc.subcore_barrier()
            return ()

        lax.fori_loop(0, 30, pair, ())

        fire_fill(wh, 61 * _S, 512, slab_b, tcond)
        gather_slab(jnp.int32(60), slab_a, ())
        wait_fill(wh, 512, slab_b, tcond)
        plsc.subcore_barrier()
        gather_slab(jnp.int32(61), slab_b, ())

        st30, cn30 = _bin_bounds(jnp.int32(61))

        def tgrp(g, _):
            off = st30 + g * 16
            pk = binned[pl.ds(off, 16)]
            lr0 = pk & (_S - 1)
            bpvec = lax.shift_right_logical(pk, 14)
            msk = (iot < (cn30 - g * 16)) & (lr0 >= 512)
            col = jnp.where(msk, lr0 - 512, 0)
            for c in range(_D):
                vals = plsc.load_gather(
                    tailb, [jnp.full((16,), c, _i32), col])
                plsc.store_scatter(
                    stage, [jnp.full((16,), c, _i32), bpvec], vals, mask=msk)
            return ()

        lax.fori_loop(0, (cn30 + 15) // 16, tgrp, ())

        off = pl.multiple_of(slot * _B + sid * _BPT, 128)
        pltpu.sync_copy(stage, out.at[:, pl.ds(off, _BPT)])

    for slot, (ih, wh, wt) in enumerate(
            ((i0, w0, t0), (i1, w1, t1), (i2, w2, t2), (i3, w3, t3))):
        @pl.when(cid == slot // 2)
        def _():
            process(ih, wh, wt, slot)


def kernel(idx0, idx1, idx2, idx3, W0, W1, W2, W3):
    tails = [W[_V - 64:, :].T for W in (W0, W1, W2, W3)]
    out_t = _grouped_gather(
        idx0, idx1, idx2, idx3, W0.T, W1.T, W2.T, W3.T, *tails
    )
    return out_t.T


# speedup vs baseline: 1.0363x; 1.0363x over previous
import functools

import jax
import jax.numpy as jnp
from jax import lax
from jax.experimental import pallas as pl
from jax.experimental.pallas import tpu as pltpu
from jax.experimental.pallas import tpu_sc as plsc

_V = 1000000
_D = 16
_B = 81920
_S = 16384
_NSLAB = 62
_BPT = _B // 16
_G = _BPT // 16

_info = plsc.get_sparse_core_info()
_NC = _info.num_cores
_NS = _info.num_subcores

_mesh = plsc.VectorSubcoreMesh(core_axis_name="c", subcore_axis_name="s")

_i32 = jnp.int32


def _lane_gather(x, i):
    dn = lax.GatherDimensionNumbers(
        offset_dims=(), collapsed_slice_dims=(0,), start_index_map=(0,)
    )
    return lax.gather(
        x, i[:, None], dimension_numbers=dn, slice_sizes=(1,),
        mode=lax.GatherScatterMode.PROMISE_IN_BOUNDS,
    )


@functools.partial(
    pl.kernel,
    mesh=_mesh,
    compiler_params=pltpu.CompilerParams(needs_layout_passes=False),
    out_type=jax.ShapeDtypeStruct((_D, 4 * _B), jnp.float32),
    scratch_types=[
        pltpu.VMEM((_BPT,), _i32),
        pltpu.VMEM((_BPT,), _i32),
        pltpu.VMEM((64,), _i32),
        pltpu.VMEM((64,), _i32),
        pltpu.VMEM((64,), _i32),
        pltpu.VMEM((1024,), jnp.float32),
        pltpu.VMEM((_D, _BPT), jnp.float32),
        pltpu.VMEM((_D, 64), jnp.float32),
        pltpu.VMEM_SHARED((_D * _S,), jnp.float32),
        pltpu.VMEM_SHARED((_D * _S,), jnp.float32),
        pltpu.SemaphoreType.DMA,
        pltpu.SemaphoreType.DMA,
    ],
)
def _grouped_gather(i0, i1, i2, i3, w0, w1, w2, w3, t0, t1, t2, t3, out,
                    idx_v, binned, cnt, fill, starts, cbuf,
                    stage, tailb, slab_a, slab_b, gsem, fsem):
    cid = lax.axis_index("c")
    sid = lax.axis_index("s")
    iot = lax.iota(_i32, 16)

    def fire_fill(wh, r0, sz, buf, cond):
        for c in range(_D):
            @pl.when((sid == c) & cond)
            def _():
                pltpu.async_copy(
                    wh.at[c, pl.ds(r0, sz)], buf.at[pl.ds(c * _S, sz)], fsem)

    def wait_fill(wh, sz, buf, cond):
        @pl.when(cond)
        def _():
            pltpu.make_async_copy(
                wh.at[0, pl.ds(0, sz)], buf.at[pl.ds(0, sz)], fsem).wait()

    def seg_rank(keys):
        bins = lax.shift_right_logical(keys, 14)
        prev = _lane_gather(bins, jnp.maximum(iot - 1, 0))
        newrun = (bins != prev) | (iot == 0)
        runstart = plsc.cummax(jnp.where(newrun, iot, 0))
        rank = iot - runstart
        flagv = jnp.where(newrun, 1, 0)
        nxt = _lane_gather(flagv, jnp.minimum(iot + 1, 15))
        last = (nxt == 1) | (iot == 15)
        return bins, rank, last

    def _bin_bounds(s):
        lane = s % 16
        half = s // 16
        sv = starts[pl.ds(0, 16)]
        cv = cnt[pl.ds(0, 16)]
        for h in range(1, 4):
            sv = jnp.where(half == h, starts[pl.ds(16 * h, 16)], sv)
            cv = jnp.where(half == h, cnt[pl.ds(16 * h, 16)], cv)
        st = jnp.sum(jnp.where(iot == lane, sv, 0))
        cn = jnp.sum(jnp.where(iot == lane, cv, 0))
        st = jnp.clip(st, 0, _BPT - 1)
        cn = jnp.clip(cn, 0, _BPT - st)
        return st, cn

    def gather_slab(s, buf, wait_handles):
        st, cn = _bin_bounds(s)
        r0 = s * _S
        ng = (cn + 15) // 16

        def fire(u):
            off = st + u * 16
            pk = binned[pl.ds(off, 16)]
            lr0 = pk & (_S - 1)
            msk = (iot < (cn - u * 16)) & ((r0 + lr0) < _V - 64)
            lr = jnp.where(msk, lr0, 0)
            half = (u & 3) * 256
            for c in range(_D):
                pltpu.async_copy(
                    buf.at[lr + c * _S],
                    cbuf.at[pl.ds(half + c * 16, 16)], gsem)

        @pl.when(ng > 0)
        def _():
            fire(jnp.int32(0))
        @pl.when(ng > 1)
        def _():
            fire(jnp.int32(1))
        @pl.when(ng > 2)
        def _():
            fire(jnp.int32(2))

        def grp(u, _):
            @pl.when(u + 3 < ng)
            def _():
                fire(u + 3)
            pltpu.make_async_copy(
                buf.at[pl.ds(0, 256)], cbuf.at[pl.ds(0, 256)], gsem).wait()
            off = st + u * 16
            pk = binned[pl.ds(off, 16)]
            bpvec = lax.shift_right_logical(pk, 14)
            lr0 = pk & (_S - 1)
            msk = (iot < (cn - u * 16)) & ((r0 + lr0) < _V - 64)
            half = (u & 3) * 256
            for c in range(_D):
                vals = cbuf[pl.ds(half + c * 16, 16)]
                plsc.store_scatter(
                    stage, [jnp.full((16,), c, _i32), bpvec], vals, mask=msk)
            return ()

        lax.fori_loop(0, ng, grp, ())
        for h in wait_handles:
            h.wait()

    def process(ih, wh, wt, slot):
        plsc.subcore_barrier()
        pltpu.sync_copy(ih.at[pl.ds(sid * _BPT, _BPT)], idx_v)
        pltpu.sync_copy(wt, tailb)
        z16 = jnp.zeros((16,), _i32)
        for h in range(4):
            cnt[pl.ds(16 * h, 16)] = z16
            fill[pl.ds(16 * h, 16)] = z16

        rprobe, _unused = plsc.scan_count(jnp.zeros((16,), _i32))
        rbase = jnp.sum(jnp.where(iot == 0, rprobe, 0))

        def pass_a(g, _):
            off = g * 16
            rv = idx_v[pl.ds(off, 16)]
            bins = lax.shift_right_logical(rv, 14)
            rank, last = plsc.scan_count(bins)
            rank = rank - rbase
            plsc.addupdate_scatter(cnt, [bins], rank + 1, mask=last)
            return ()

        lax.fori_loop(0, _G, pass_a, ())

        carry = jnp.int32(0)
        for h in range(4):
            ch = cnt[pl.ds(16 * h, 16)]
            inh = plsc.cumsum(ch) + carry
            starts[pl.ds(16 * h, 16)] = inh - ch
            carry = carry + jnp.sum(ch)

        def pass_b(g, _):
            off = g * 16
            rv = idx_v[pl.ds(off, 16)]
            bp = iot + off
            bins = lax.shift_right_logical(rv, 14)
            rank, last = plsc.scan_count(bins)
            rank = rank - rbase
            stv = plsc.load_gather(starts, [bins])
            fl = plsc.load_gather(fill, [bins])
            pos = jnp.clip(stv + fl + rank, 0, _BPT - 1)
            pk = lax.shift_left(bp, 14) | (rv & (_S - 1))
            plsc.store_scatter(binned, [pos], pk)
            plsc.addupdate_scatter(fill, [bins], rank + 1, mask=last)
            return ()

        lax.fori_loop(0, _G, pass_b, ())

        tcond = jnp.bool_(True)
        fire_fill(wh, 0, _S, slab_a, tcond)
        wait_fill(wh, _S, slab_a, tcond)
        plsc.subcore_barrier()

        def pair(u, _):
            r0b = pl.multiple_of((2 * u + 1) * _S, 128)
            fire_fill(wh, r0b, _S, slab_b, tcond)
            gather_slab(2 * u, slab_a, ())
            wait_fill(wh, _S, slab_b, tcond)
            plsc.subcore_barrier()
            s_a = 2 * u + 2
            do_fill = s_a <= 60
            r0a = pl.multiple_of(s_a * _S, 128)
            fire_fill(wh, r0a, _S, slab_a, do_fill)
            gather_slab(2 * u + 1, slab_b, ())
            wait_fill(wh, _S, slab_a, do_fill)
            plsc.subcore_barrier()
            return ()

        lax.fori_loop(0, 30, pair, ())

        fire_fill(wh, 61 * _S, 512, slab_b, tcond)
        gather_slab(jnp.int32(60), slab_a, ())
        wait_fill(wh, 512, slab_b, tcond)
        plsc.subcore_barrier()
        gather_slab(jnp.int32(61), slab_b, ())

        st30, cn30 = _bin_bounds(jnp.int32(61))

        def tgrp(g, _):
            off = st30 + g * 16
            pk = binned[pl.ds(off, 16)]
            lr0 = pk & (_S - 1)
            bpvec = lax.shift_right_logical(pk, 14)
            msk = (iot < (cn30 - g * 16)) & (lr0 >= 512)
            col = jnp.where(msk, lr0 - 512, 0)
            for c in range(_D):
                vals = plsc.load_gather(
                    tailb, [jnp.full((16,), c, _i32), col])
                plsc.store_scatter(
                    stage, [jnp.full((16,), c, _i32), bpvec], vals, mask=msk)
            return ()

        lax.fori_loop(0, (cn30 + 15) // 16, tgrp, ())

        off = pl.multiple_of(slot * _B + sid * _BPT, 128)
        pltpu.sync_copy(stage, out.at[:, pl.ds(off, _BPT)])

    for slot, (ih, wh, wt) in enumerate(
            ((i0, w0, t0), (i1, w1, t1), (i2, w2, t2), (i3, w3, t3))):
        @pl.when(cid == slot // 2)
        def _():
            process(ih, wh, wt, slot)


def kernel(idx0, idx1, idx2, idx3, W0, W1, W2, W3):
    tails = [W[_V - 64:, :].T for W in (W0, W1, W2, W3)]
    out_t = _grouped_gather(
        idx0, idx1, idx2, idx3, W0.T, W1.T, W2.T, W3.T, *tails
    )
    return out_t.T


# speedup vs baseline: 1.0377x; 1.0013x over previous
import functools

import jax
import jax.numpy as jnp
from jax import lax
from jax.experimental import pallas as pl
from jax.experimental.pallas import tpu as pltpu
from jax.experimental.pallas import tpu_sc as plsc

_V = 1000000
_D = 16
_B = 81920
_S = 16384
_NSLAB = 62
_BPT = _B // 16
_G = _BPT // 16

_info = plsc.get_sparse_core_info()
_NC = _info.num_cores
_NS = _info.num_subcores

_mesh = plsc.VectorSubcoreMesh(core_axis_name="c", subcore_axis_name="s")

_i32 = jnp.int32


@functools.partial(
    pl.kernel,
    mesh=_mesh,
    compiler_params=pltpu.CompilerParams(needs_layout_passes=False),
    out_type=jax.ShapeDtypeStruct((_D, 4 * _B), jnp.float32),
    scratch_types=[
        pltpu.VMEM((_BPT,), _i32),
        pltpu.VMEM((_BPT,), _i32),
        pltpu.VMEM((64,), _i32),
        pltpu.VMEM((64,), _i32),
        pltpu.VMEM((64,), _i32),
        pltpu.VMEM((1024,), jnp.float32),
        pltpu.VMEM((_D, _BPT), jnp.float32),
        pltpu.VMEM((_D, 64), jnp.float32),
        pltpu.VMEM_SHARED((_D * _S,), jnp.float32),
        pltpu.VMEM_SHARED((_D * _S,), jnp.float32),
        pltpu.SemaphoreType.DMA,
        pltpu.SemaphoreType.DMA,
    ],
)
def _grouped_gather(i0, i1, i2, i3, w0, w1, w2, w3, t0, t1, t2, t3, out,
                    idx_v, binned, cnt, fill, starts, cbuf,
                    stage, tailb, slab_a, slab_b, gsem, fsem):
    cid = lax.axis_index("c")
    sid = lax.axis_index("s")
    iot = lax.iota(_i32, 16)

    def fire_fill(wh, r0, sz, buf, cond):
        for c in range(_D):
            @pl.when((sid == c) & cond)
            def _():
                pltpu.async_copy(
                    wh.at[c, pl.ds(r0, sz)], buf.at[pl.ds(c * _S, sz)], fsem)

    def wait_fill(wh, sz, buf, cond):
        @pl.when(cond)
        def _():
            pltpu.make_async_copy(
                wh.at[0, pl.ds(0, sz)], buf.at[pl.ds(0, sz)], fsem).wait()

    def _bin_bounds(s):
        lane = s % 16
        half = s // 16
        sv = starts[pl.ds(0, 16)]
        cv = cnt[pl.ds(0, 16)]
        for h in range(1, 4):
            sv = jnp.where(half == h, starts[pl.ds(16 * h, 16)], sv)
            cv = jnp.where(half == h, cnt[pl.ds(16 * h, 16)], cv)
        st = jnp.sum(jnp.where(iot == lane, sv, 0))
        cn = jnp.sum(jnp.where(iot == lane, cv, 0))
        st = jnp.clip(st, 0, _BPT - 1)
        cn = jnp.clip(cn, 0, _BPT - st)
        return st, cn

    def gather_slab(s, buf, wait_handles):
        st, cn = _bin_bounds(s)
        r0 = s * _S
        ng = (cn + 15) // 16

        def fire(u):
            off = st + u * 16
            pk = binned[pl.ds(off, 16)]
            lr0 = pk & (_S - 1)
            msk = (iot < (cn - u * 16)) & ((r0 + lr0) < _V - 64)
            lr = jnp.where(msk, lr0, 0)
            half = (u & 3) * 256
            for c in range(_D):
                pltpu.async_copy(
                    buf.at[lr + c * _S],
                    cbuf.at[pl.ds(half + c * 16, 16)], gsem)

        @pl.when(ng > 0)
        def _():
            fire(jnp.int32(0))
        @pl.when(ng > 1)
        def _():
            fire(jnp.int32(1))
        @pl.when(ng > 2)
        def _():
            fire(jnp.int32(2))

        def grp(u, _):
            @pl.when(u + 3 < ng)
            def _():
                fire(u + 3)
            pltpu.make_async_copy(
                buf.at[pl.ds(0, 256)], cbuf.at[pl.ds(0, 256)], gsem).wait()
            off = st + u * 16
            pk = binned[pl.ds(off, 16)]
            bpvec = lax.shift_right_logical(pk, 14)
            lr0 = pk & (_S - 1)
            msk = (iot < (cn - u * 16)) & ((r0 + lr0) < _V - 64)
            half = (u & 3) * 256
            for c in range(_D):
                vals = cbuf[pl.ds(half + c * 16, 16)]
                plsc.store_scatter(
                    stage, [jnp.full((16,), c, _i32), bpvec], vals, mask=msk)
            return ()

        lax.fori_loop(0, ng, grp, ())
        for h in wait_handles:
            h.wait()

    def process(ih, wh, wt, slot):
        plsc.subcore_barrier()
        pltpu.sync_copy(ih.at[pl.ds(sid * _BPT, _BPT)], idx_v)
        pltpu.sync_copy(wt, tailb)
        z16 = jnp.zeros((16,), _i32)
        for h in range(4):
            cnt[pl.ds(16 * h, 16)] = z16
            fill[pl.ds(16 * h, 16)] = z16

        rprobe, _unused = plsc.scan_count(jnp.zeros((16,), _i32))
        rbase = jnp.sum(jnp.where(iot == 0, rprobe, 0))

        def pass_a(g, _):
            off = g * 16
            rv = idx_v[pl.ds(off, 16)]
            bins = lax.shift_right_logical(rv, 14)
            rank, last = plsc.scan_count(bins)
            rank = rank - rbase
            plsc.addupdate_scatter(cnt, [bins], rank + 1, mask=last)
            return ()

        lax.fori_loop(0, _G, pass_a, ())

        carry = jnp.int32(0)
        for h in range(4):
            ch = cnt[pl.ds(16 * h, 16)]
            inh = plsc.cumsum(ch) + carry
            starts[pl.ds(16 * h, 16)] = inh - ch
            carry = carry + jnp.sum(ch)

        def pass_b(g, _):
            off = g * 16
            rv = idx_v[pl.ds(off, 16)]
            bp = iot + off
            bins = lax.shift_right_logical(rv, 14)
            rank, last = plsc.scan_count(bins)
            rank = rank - rbase
            stv = plsc.load_gather(starts, [bins])
            fl = plsc.load_gather(fill, [bins])
            pos = jnp.clip(stv + fl + rank, 0, _BPT - 1)
            pk = lax.shift_left(bp, 14) | (rv & (_S - 1))
            plsc.store_scatter(binned, [pos], pk)
            plsc.addupdate_scatter(fill, [bins], rank + 1, mask=last)
            return ()

        lax.fori_loop(0, _G, pass_b, ())

        tcond = jnp.bool_(True)
        fire_fill(wh, 0, _S, slab_a, tcond)
        wait_fill(wh, _S, slab_a, tcond)
        plsc.subcore_barrier()

        def pair(u, _):
            r0b = pl.multiple_of((2 * u + 1) * _S, 128)
            fire_fill(wh, r0b, _S, slab_b, tcond)
            gather_slab(2 * u, slab_a, ())
            wait_fill(wh, _S, slab_b, tcond)
            plsc.subcore_barrier()
            s_a = 2 * u + 2
            do_fill = s_a <= 60
            r0a = pl.multiple_of(s_a * _S, 128)
            fire_fill(wh, r0a, _S, slab_a, do_fill)
            gather_slab(2 * u + 1, slab_b, ())
            wait_fill(wh, _S, slab_a, do_fill)
            plsc.subcore_barrier()
            return ()

        lax.fori_loop(0, 30, pair, ())

        fire_fill(wh, 61 * _S, 512, slab_b, tcond)
        gather_slab(jnp.int32(60), slab_a, ())
        wait_fill(wh, 512, slab_b, tcond)
        plsc.subcore_barrier()
        gather_slab(jnp.int32(61), slab_b, ())

        st30, cn30 = _bin_bounds(jnp.int32(61))

        def tgrp(g, _):
            off = st30 + g * 16
            pk = binned[pl.ds(off, 16)]
            lr0 = pk & (_S - 1)
            bpvec = lax.shift_right_logical(pk, 14)
            msk = (iot < (cn30 - g * 16)) & (lr0 >= 512)
            col = jnp.where(msk, lr0 - 512, 0)
            for c in range(_D):
                vals = plsc.load_gather(
                    tailb, [jnp.full((16,), c, _i32), col])
                plsc.store_scatter(
                    stage, [jnp.full((16,), c, _i32), bpvec], vals, mask=msk)
            return ()

        lax.fori_loop(0, (cn30 + 15) // 16, tgrp, ())

        off = pl.multiple_of(slot * _B + sid * _BPT, 128)
        pltpu.sync_copy(stage, out.at[:, pl.ds(off, _BPT)])

    for slot, (ih, wh, wt) in enumerate(
            ((i0, w0, t0), (i1, w1, t1), (i2, w2, t2), (i3, w3, t3))):
        @pl.when(cid == slot // 2)
        def _():
            process(ih, wh, wt, slot)


def kernel(idx0, idx1, idx2, idx3, W0, W1, W2, W3):
    tails = [W[_V - 64:, :].T for W in (W0, W1, W2, W3)]
    out_t = _grouped_gather(
        idx0, idx1, idx2, idx3, W0.T, W1.T, W2.T, W3.T, *tails
    )
    return out_t.T
